# baseline (speedup 1.0000x reference)
import functools

import jax
import jax.numpy as jnp
from jax import lax
from jax.experimental import pallas as pl
from jax.experimental.pallas import tpu as pltpu
from jax.experimental.pallas import tpu_sc as plsc

N = 10000
NP = 10240
D = 128
E = 320000
EP = 327680
G = 100
NC = 2
NS = 16
NW = NC * NS
CH = 128
NCHUNK = 80
PCH = 40
RPT = NP // NS
EPS = 1e-5

_mesh = plsc.VectorSubcoreMesh(
    core_axis_name="c", subcore_axis_name="s", num_cores=NC, num_subcores=NS)

_HIGH = jax.lax.Precision.HIGHEST


def _dot(a, b):
    return jnp.dot(a, b, preferred_element_type=jnp.float32, precision=_HIGH)


def _deg_body(dst_h, out_h, dall, hist, tmp, shared):
    c = lax.axis_index("c")
    s = lax.axis_index("s")
    wid = c * NS + s
    zero16 = jnp.zeros((16,), jnp.float32)
    one16 = jnp.ones((16,), jnp.float32)

    def zbody(j, car):
        hist[pl.ds(j * 16, 16)] = zero16
        return car
    lax.fori_loop(0, NP // 16, zbody, 0)

    pltpu.sync_copy(dst_h.at[wid], dall)

    def hbody(j, car):
        for m in range(8):
            plsc.addupdate_scatter(hist, [dall[j, pl.ds(m * 16, 16)]], one16)
        return car
    lax.fori_loop(0, NCHUNK * CH // 128, hbody, 0)

    pltpu.sync_copy(hist, shared.at[s])
    plsc.subcore_barrier()

    off = pl.multiple_of(s * RPT, 8)
    for t in range(NS):
        pltpu.sync_copy(shared.at[t, pl.ds(off, RPT)], tmp.at[t])

    def rbody(j, car):
        acc = zero16
        for t in range(NS):
            acc = acc + tmp[t, pl.ds(j * 16, 16)]
        hist[pl.ds(off + j * 16, 16)] = acc
        return car
    lax.fori_loop(0, RPT // 16, rbody, 0)
    pltpu.sync_copy(hist.at[pl.ds(off, RPT)], out_h.at[c, pl.ds(off, RPT)])


@functools.partial(
    pl.kernel,
    out_type=jax.ShapeDtypeStruct((NC, NP), jnp.float32),
    mesh=_mesh,
    scratch_types=[
        pltpu.VMEM((NCHUNK * CH // 128, 128), jnp.int32),
        pltpu.VMEM((NP,), jnp.float32),
        pltpu.VMEM((NS, RPT), jnp.float32),
        pltpu.VMEM_SHARED((NS, NP), jnp.float32),
    ],
    compiler_params=pltpu.CompilerParams(needs_layout_passes=False),
)
def _sc_deg(dst_h, out_h, dall, hist, tmp, shared):
    _deg_body(dst_h, out_h, dall, hist, tmp, shared)


def _spmm_body(y_h, src_h, dst_h, out_h, sall, dall, rA, rB, acc,
               sgA, sgB, ssA, ssB):
    c = lax.axis_index("c")
    s = lax.axis_index("s")
    wid = c * NS + s

    zvec = jnp.zeros((16,), jnp.float32)

    def zb(r, car):
        for m in range(8):
            rA[r, pl.ds(m * 16, 16)] = zvec
        return car
    lax.fori_loop(0, CH, zb, 0)
    zoff = pl.multiple_of(s * RPT, 8)
    for q in range(RPT // CH):
        pltpu.sync_copy(rA, acc.at[pl.ds(zoff + q * CH, CH)])

    plsc.subcore_barrier()

    def fire_g(j, buf, sem):
        pltpu.async_copy(y_h.at[sall.at[j]], buf, sem)

    def wait_g(buf, sem):
        pltpu.make_async_copy(y_h.at[sall.at[0]], buf, sem).wait()

    def fire_s(j, buf, sem):
        pltpu.async_copy(buf, acc.at[dall.at[j]], sem, add=True)

    def wait_s(buf, sem):
        pltpu.make_async_copy(buf, acc.at[dall.at[0]], sem).wait()

    for p in range(NCHUNK // PCH):
        pltpu.sync_copy(src_h.at[wid, pl.ds(p * PCH, PCH)], sall)
        pltpu.sync_copy(dst_h.at[wid, pl.ds(p * PCH, PCH)], dall)
        fire_g(0, rA, sgA)

        fire_g(1, rB, sgB)

        def body(k, car):
            wait_g(rA, sgA)
            fire_s(2 * k, rA, ssA)
            wait_g(rB, sgB)
            fire_s(2 * k + 1, rB, ssB)
            wait_s(rA, ssA)
            fire_g(2 * k + 2, rA, sgA)
            wait_s(rB, ssB)
            fire_g(2 * k + 3, rB, sgB)
            return car

        lax.fori_loop(0, PCH // 2 - 1, body, 0)
        wait_g(rA, sgA)
        fire_s(PCH - 2, rA, ssA)
        wait_g(rB, sgB)
        fire_s(PCH - 1, rB, ssB)
        wait_s(rA, ssA)
        wait_s(rB, ssB)

    plsc.subcore_barrier()
    off = pl.multiple_of(s * RPT, 8)
    pltpu.sync_copy(acc.at[pl.ds(off, RPT)], out_h.at[c, pl.ds(off, RPT)])


@functools.partial(
    pl.kernel,
    out_type=jax.ShapeDtypeStruct((NC, NP, D), jnp.float32),
    mesh=_mesh,
    scratch_types=[
        pltpu.VMEM((PCH, CH), jnp.int32),
        pltpu.VMEM((PCH, CH), jnp.int32),
        pltpu.VMEM((CH, D), jnp.float32),
        pltpu.VMEM((CH, D), jnp.float32),
        pltpu.VMEM_SHARED((NP, D), jnp.float32),
        pltpu.SemaphoreType.DMA,
        pltpu.SemaphoreType.DMA,
        pltpu.SemaphoreType.DMA,
        pltpu.SemaphoreType.DMA,
    ],
)
def _sc_spmm(y_h, src_h, dst_h, out_h, sall, dall, rA, rB, acc,
             sgA, sgB, ssA, ssB):
    _spmm_body(y_h, src_h, dst_h, out_h, sall, dall, rA, rB, acc,
               sgA, sgB, ssA, ssB)


def _mm_body(x_ref, w_ref, z_ref):
    z_ref[...] = _dot(x_ref[...], w_ref[...])


def _scale_body(z_ref, d0_ref, d1_ref, y_ref, dinv_ref):
    deg = d0_ref[...] + d1_ref[...] + 1.0
    dinv = lax.rsqrt(jnp.maximum(deg, 1.0))
    y_ref[pl.ds(0, N), :] = z_ref[...] * dinv
    y_ref[pl.ds(N, NP - N), :] = jnp.zeros((NP - N, D), jnp.float32)
    dinv_ref[...] = dinv


def _bn_relu(t, g, be):
    m = jnp.mean(t, axis=0, keepdims=True)
    tc = t - m
    v = jnp.mean(tc * tc, axis=0, keepdims=True)
    return jnp.maximum(tc * lax.rsqrt(v + EPS) * g + be, 0.0)


def _mid_body(p_ref, y_ref, dinv_ref, g_ref, be_ref, w_ref, out_ref):
    dinv = dinv_ref[...]
    t = (p_ref[0, pl.ds(0, N), :] + p_ref[1, pl.ds(0, N), :]
         + y_ref[pl.ds(0, N), :]) * dinv
    t = _bn_relu(t, g_ref[...], be_ref[...])
    out_ref[pl.ds(0, N), :] = _dot(t, w_ref[...]) * dinv
    out_ref[pl.ds(N, NP - N), :] = jnp.zeros((NP - N, D), jnp.float32)


def _head_body(p_ref, y_ref, dinv_ref, g_ref, be_ref, bat_ref,
               wm1_ref, gm_ref, bem_ref, wm2_ref, bm2_ref, out_ref):
    t = (p_ref[0, pl.ds(0, N), :] + p_ref[1, pl.ds(0, N), :]
         + y_ref[pl.ds(0, N), :]) * dinv_ref[...]
    h = _bn_relu(t, g_ref[...], be_ref[...])
    gid = lax.broadcasted_iota(jnp.int32, (G, N), 0)
    m = (gid == bat_ref[...]).astype(jnp.float32)
    ssum = _dot(m, h)
    cnt = jnp.sum(m, axis=1, keepdims=True)
    pooled = ssum / jnp.maximum(cnt, 1.0)
    q = _bn_relu(_dot(pooled, wm1_ref[...]), gm_ref[...], bem_ref[...])
    logits = _dot(q, wm2_ref[...]) + bm2_ref[...]
    out_ref[...] = 1.0 / (1.0 + jnp.exp(-logits))


def _tc_call(body, out_shape):
    return pl.pallas_call(body, out_shape=out_shape)


def kernel(features, edge_index, batch, W1, b1, g1, be1, W2, b2, g2, be2,
           Wm1, bm1, gm, bem, Wm2, bm2):
    x = features.reshape(N, D)
    pad = N + jnp.arange(EP - E, dtype=edge_index.dtype) % (NP - N)
    srcf = jnp.concatenate([edge_index[0], pad])
    dstf = jnp.concatenate([edge_index[1], pad])
    src = srcf.reshape(NW, NCHUNK, CH)
    dst = dstf.reshape(NW, NCHUNK, CH)
    dst16 = dstf.reshape(NW, NCHUNK * CH // 128, 128)

    degp = _sc_deg(dst16)

    z1 = _tc_call(_mm_body, jax.ShapeDtypeStruct((N, D), jnp.float32))(x, W1)

    y1, dinv = _tc_call(
        _scale_body,
        (jax.ShapeDtypeStruct((NP, D), jnp.float32),
         jax.ShapeDtypeStruct((N, 1), jnp.float32)),
    )(z1, degp[0, :N, None], degp[1, :N, None])

    p1 = _sc_spmm(y1, src, dst)

    y2 = _tc_call(_mid_body, jax.ShapeDtypeStruct((NP, D), jnp.float32))(
        p1, y1, dinv, g1.reshape(1, D), be1.reshape(1, D), W2)

    p2 = _sc_spmm(y2, src, dst)

    out = _tc_call(_head_body, jax.ShapeDtypeStruct((G, 16), jnp.float32))(
        p2, y2, dinv, g2.reshape(1, D), be2.reshape(1, D),
        batch.reshape(1, N), Wm1, gm.reshape(1, D), bem.reshape(1, D),
        Wm2, bm2.reshape(1, 16))
    return out


# speedup vs baseline: 1.2299x; 1.2299x over previous
import functools

import jax
import jax.numpy as jnp
from jax import lax
from jax.experimental import pallas as pl
from jax.experimental.pallas import tpu as pltpu
from jax.experimental.pallas import tpu_sc as plsc

N = 10000
NP = 10240
D = 128
E = 320000
EP = 327680
G = 100
NC = 2
NS = 16
NW = NC * NS
CH = 128
NCHUNK = 80
PCH = 40
RPT = NP // NS
EPS = 1e-5

_mesh = plsc.VectorSubcoreMesh(
    core_axis_name="c", subcore_axis_name="s", num_cores=NC, num_subcores=NS)

_HIGH = jax.lax.Precision.HIGHEST


def _dot(a, b):
    return jnp.dot(a, b, preferred_element_type=jnp.float32, precision=_HIGH)


def _deg_body(dst_h, out_h, dall, hist, tmp, shared):
    c = lax.axis_index("c")
    s = lax.axis_index("s")
    wid = c * NS + s
    zero16 = jnp.zeros((16,), jnp.float32)
    one16 = jnp.ones((16,), jnp.float32)

    def zbody(j, car):
        hist[pl.ds(j * 16, 16)] = zero16
        return car
    lax.fori_loop(0, NP // 16, zbody, 0)

    pltpu.sync_copy(dst_h.at[wid], dall)

    def hbody(j, car):
        for m in range(8):
            plsc.addupdate_scatter(hist, [dall[j, pl.ds(m * 16, 16)]], one16)
        return car
    lax.fori_loop(0, NCHUNK * CH // 128, hbody, 0)

    pltpu.sync_copy(hist, shared.at[s])
    plsc.subcore_barrier()

    off = pl.multiple_of(s * RPT, 8)
    for t in range(NS):
        pltpu.sync_copy(shared.at[t, pl.ds(off, RPT)], tmp.at[t])

    def rbody(j, car):
        acc = zero16
        for t in range(NS):
            acc = acc + tmp[t, pl.ds(j * 16, 16)]
        hist[pl.ds(off + j * 16, 16)] = acc
        return car
    lax.fori_loop(0, RPT // 16, rbody, 0)
    pltpu.sync_copy(hist.at[pl.ds(off, RPT)], out_h.at[c, pl.ds(off, RPT)])


@functools.partial(
    pl.kernel,
    out_type=jax.ShapeDtypeStruct((NC, NP), jnp.float32),
    mesh=_mesh,
    scratch_types=[
        pltpu.VMEM((NCHUNK * CH // 128, 128), jnp.int32),
        pltpu.VMEM((NP,), jnp.float32),
        pltpu.VMEM((NS, RPT), jnp.float32),
        pltpu.VMEM_SHARED((NS, NP), jnp.float32),
    ],
    compiler_params=pltpu.CompilerParams(needs_layout_passes=False),
)
def _sc_deg(dst_h, out_h, dall, hist, tmp, shared):
    _deg_body(dst_h, out_h, dall, hist, tmp, shared)


def _spmm_body(y_h, src_h, dst_h, out_h, sall, dall, rA, rB, acc,
               sgA, sgB, ssA, ssB):
    c = lax.axis_index("c")
    s = lax.axis_index("s")
    wid = c * NS + s

    zvec = jnp.zeros((16,), jnp.float32)

    def zb(r, car):
        for m in range(8):
            rA[r, pl.ds(m * 16, 16)] = zvec
        return car
    lax.fori_loop(0, CH, zb, 0)
    zoff = pl.multiple_of(s * RPT, 8)
    for q in range(RPT // CH):
        pltpu.sync_copy(rA, acc.at[pl.ds(zoff + q * CH, CH)])

    plsc.subcore_barrier()

    def fire_g(j, buf, sem):
        pltpu.async_copy(y_h.at[sall.at[j]], buf, sem)

    def wait_g(buf, sem):
        pltpu.make_async_copy(y_h.at[sall.at[0]], buf, sem).wait()

    def fire_s(j, buf, sem):
        pltpu.async_copy(buf, acc.at[dall.at[j]], sem, add=True)

    def wait_s(buf, sem):
        pltpu.make_async_copy(buf, acc.at[dall.at[0]], sem).wait()

    for p in range(NCHUNK // PCH):
        pltpu.sync_copy(src_h.at[wid, pl.ds(p * PCH, PCH)], sall)
        pltpu.sync_copy(dst_h.at[wid, pl.ds(p * PCH, PCH)], dall)
        fire_g(0, rA, sgA)

        def body(k, car):
            fire_g(2 * k + 1, rB, sgB)
            wait_g(rA, sgA)
            fire_s(2 * k, rA, ssA)
            wait_s(rA, ssA)

            @pl.when(k < PCH // 2 - 1)
            def _():
                fire_g(2 * k + 2, rA, sgA)

            wait_g(rB, sgB)
            fire_s(2 * k + 1, rB, ssB)
            wait_s(rB, ssB)
            return car

        lax.fori_loop(0, PCH // 2, body, 0)

    plsc.subcore_barrier()
    off = pl.multiple_of(s * RPT, 8)
    pltpu.sync_copy(acc.at[pl.ds(off, RPT)], out_h.at[c, pl.ds(off, RPT)])


@functools.partial(
    pl.kernel,
    out_type=jax.ShapeDtypeStruct((NC, NP, D), jnp.float32),
    mesh=_mesh,
    scratch_types=[
        pltpu.VMEM((PCH, CH), jnp.int32),
        pltpu.VMEM((PCH, CH), jnp.int32),
        pltpu.VMEM((CH, D), jnp.float32),
        pltpu.VMEM((CH, D), jnp.float32),
        pltpu.VMEM_SHARED((NP, D), jnp.float32),
        pltpu.SemaphoreType.DMA,
        pltpu.SemaphoreType.DMA,
        pltpu.SemaphoreType.DMA,
        pltpu.SemaphoreType.DMA,
    ],
)
def _sc_spmm(y_h, src_h, dst_h, out_h, sall, dall, rA, rB, acc,
             sgA, sgB, ssA, ssB):
    _spmm_body(y_h, src_h, dst_h, out_h, sall, dall, rA, rB, acc,
               sgA, sgB, ssA, ssB)


def _lin1_body(x_ref, w_ref, d0_ref, d1_ref, y_ref, dinv_ref):
    deg = d0_ref[...] + d1_ref[...] + 1.0
    dinv = lax.rsqrt(jnp.maximum(deg, 1.0))
    y_ref[pl.ds(0, N), :] = _dot(x_ref[...], w_ref[...]) * dinv
    y_ref[pl.ds(N, NP - N), :] = jnp.zeros((NP - N, D), jnp.float32)
    dinv_ref[...] = dinv


def _bn_relu(t, g, be):
    m = jnp.mean(t, axis=0, keepdims=True)
    tc = t - m
    v = jnp.mean(tc * tc, axis=0, keepdims=True)
    return jnp.maximum(tc * lax.rsqrt(v + EPS) * g + be, 0.0)


def _mid_body(p_ref, y_ref, dinv_ref, g_ref, be_ref, w_ref, out_ref):
    dinv = dinv_ref[...]
    t = (p_ref[0, pl.ds(0, N), :] + p_ref[1, pl.ds(0, N), :]
         + y_ref[pl.ds(0, N), :]) * dinv
    t = _bn_relu(t, g_ref[...], be_ref[...])
    out_ref[pl.ds(0, N), :] = _dot(t, w_ref[...]) * dinv
    out_ref[pl.ds(N, NP - N), :] = jnp.zeros((NP - N, D), jnp.float32)


def _head_body(p_ref, y_ref, dinv_ref, g_ref, be_ref, bat_ref,
               wm1_ref, gm_ref, bem_ref, wm2_ref, bm2_ref, out_ref):
    t = (p_ref[0, pl.ds(0, N), :] + p_ref[1, pl.ds(0, N), :]
         + y_ref[pl.ds(0, N), :]) * dinv_ref[...]
    h = _bn_relu(t, g_ref[...], be_ref[...])
    gid = lax.broadcasted_iota(jnp.int32, (G, N), 0)
    m = (gid == bat_ref[...]).astype(jnp.float32)
    ssum = _dot(m, h)
    cnt = jnp.sum(m, axis=1, keepdims=True)
    pooled = ssum / jnp.maximum(cnt, 1.0)
    q = _bn_relu(_dot(pooled, wm1_ref[...]), gm_ref[...], bem_ref[...])
    logits = _dot(q, wm2_ref[...]) + bm2_ref[...]
    out_ref[...] = 1.0 / (1.0 + jnp.exp(-logits))


def _tc_call(body, out_shape):
    return pl.pallas_call(body, out_shape=out_shape)


def kernel(features, edge_index, batch, W1, b1, g1, be1, W2, b2, g2, be2,
           Wm1, bm1, gm, bem, Wm2, bm2):
    x = features.reshape(N, D)
    pad = N + jnp.arange(EP - E, dtype=edge_index.dtype) % (NP - N)
    srcf = jnp.concatenate([edge_index[0], pad])
    dstf = jnp.concatenate([edge_index[1], pad])
    src = srcf.reshape(NW, NCHUNK, CH)
    dst = dstf.reshape(NW, NCHUNK, CH)
    dst16 = dstf.reshape(NW, NCHUNK * CH // 128, 128)

    degp = _sc_deg(dst16)

    y1, dinv = _tc_call(
        _lin1_body,
        (jax.ShapeDtypeStruct((NP, D), jnp.float32),
         jax.ShapeDtypeStruct((N, 1), jnp.float32)),
    )(x, W1, degp[0, :N, None], degp[1, :N, None])

    p1 = _sc_spmm(y1, src, dst)

    y2 = _tc_call(_mid_body, jax.ShapeDtypeStruct((NP, D), jnp.float32))(
        p1, y1, dinv, g1.reshape(1, D), be1.reshape(1, D), W2)

    p2 = _sc_spmm(y2, src, dst)

    out = _tc_call(_head_body, jax.ShapeDtypeStruct((G, 16), jnp.float32))(
        p2, y2, dinv, g2.reshape(1, D), be2.reshape(1, D),
        batch.reshape(1, N), Wm1, gm.reshape(1, D), bem.reshape(1, D),
        Wm2, bm2.reshape(1, 16))
    return out
